# baseline (speedup 1.0000x reference)
import functools

import jax
import jax.numpy as jnp
from jax import lax
from jax.experimental import pallas as pl
from jax.experimental.pallas import tpu as pltpu
from jax.experimental.pallas import tpu_sc as plsc

D_MODEL = 64
VOCAB = 1000000
BATCH = 4096
HIST = 50

NC = 2
NS = 16
NW = NC * NS

B_TOTAL = BATCH * HIST
B_PER_W = B_TOTAL // NW
GRP = 128
N_STAGE = B_PER_W // GRP
N_CHUNK = GRP // 16

SCALE = 8.0


def _mesh():
    return plsc.VectorSubcoreMesh(core_axis_name="c", subcore_axis_name="s")


@functools.partial(
    pl.kernel,
    mesh=_mesh(),
    out_type=jax.ShapeDtypeStruct(
        (NW, N_STAGE, GRP // 2, 2 * D_MODEL), jnp.float32),
    scratch_types=[
        pltpu.VMEM((N_STAGE, GRP), jnp.int32),
        pltpu.VMEM((2, GRP), jnp.int32),
        pltpu.VMEM((2, GRP, 2 * D_MODEL), jnp.float32),
        pltpu.VMEM((2, GRP // 2, 2 * D_MODEL), jnp.float32),
        pltpu.SemaphoreType.DMA,
        pltpu.SemaphoreType.DMA,
    ],
    compiler_params=pltpu.CompilerParams(needs_layout_passes=False),
)
def _gather_scale(idx_hbm, table_hbm, out_hbm, idx_v, idxp_v, buf, obuf,
                  sem0, sem1):
    wid = lax.axis_index("s") * NC + lax.axis_index("c")
    sems = (sem0, sem1)
    pltpu.sync_copy(idx_hbm.at[wid], idx_v)

    lanes = lax.iota(jnp.int32, 16)

    def pair_indices(st, p):
        def c_body(c, carry):
            v = idx_v[st, pl.ds(c * 16, 16)]
            idxp_v[p, pl.ds(c * 16, 16)] = lax.shift_right_logical(v, 1)
            return carry
        lax.fori_loop(0, N_CHUNK, c_body, 0)

    def fire(p):
        pltpu.async_copy(table_hbm.at[idxp_v.at[p]], buf.at[p], sems[p])

    def drain(p):
        pltpu.make_async_copy(
            table_hbm.at[idxp_v.at[p]], buf.at[p], sems[p]).wait()

    def select_scale_store(st, p):
        def c_body(c, carry):
            kvec = c * 16 + lanes
            parity = lax.bitwise_and(idx_v[st, pl.ds(c * 16, 16)], 1)
            srow = kvec
            scol = parity * D_MODEL
            drow = lax.shift_right_logical(kvec, 1)
            dcol = lax.bitwise_and(kvec, 1) * D_MODEL
            for j in range(D_MODEL):
                v = plsc.load_gather(buf.at[p], [srow, scol + j])
                plsc.store_scatter(obuf.at[p], [drow, dcol + j], v * SCALE)
            return carry
        lax.fori_loop(0, N_CHUNK, c_body, 0)
        pltpu.sync_copy(obuf.at[p], out_hbm.at[wid, st])

    def stage(st, p):
        @pl.when(st + 1 < N_STAGE)
        def _fire_next():
            pair_indices(st + 1, 1 - p)
            fire(1 - p)
        drain(p)
        select_scale_store(st, p)

    pair_indices(0, 0)
    fire(0)

    def pair_body(u, carry):
        stage(2 * u, 0)
        stage(2 * u + 1, 1)
        return carry
    lax.fori_loop(0, N_STAGE // 2, pair_body, 0)


def kernel(x, W):
    idx_t = jnp.transpose(x.reshape(BATCH, HIST)).astype(jnp.int32)
    idx = idx_t.reshape(NW, N_STAGE, GRP)
    table = W.reshape(VOCAB // 2, 2 * D_MODEL)
    out = _gather_scale(idx, table)
    return jnp.transpose(out.reshape(HIST, BATCH, D_MODEL), (1, 0, 2))
